# baseline (speedup 1.0000x reference)
import functools

import jax
import jax.numpy as jnp
from jax import lax
from jax.experimental import pallas as pl
from jax.experimental.pallas import tpu as pltpu
from jax.experimental.pallas import tpu_sc as plsc

_B, _H, _W, _NR = 4, 224, 224, 14
_BS = _H // _NR
_S = _NR * _NR
_ROWS = _B * _S
_Q = _B * _NR
_NPIX = _H * _W
_CH = [11, 96, 192, 384, 768, 768]
_VOFF = [0, 96, 288, 672, 1440]
_VTOT = 2208
_NWORKERS = 32


def _iota(shape, dim):
    return lax.broadcasted_iota(jnp.int32, shape, dim)


def _extract_sum(v):
    s = v[0]
    for k in range(1, 16):
        s = s + v[k]
    return s


def _sc_strip(x_hbm, sums_hbm, seg_hbm, rows_v, row1_v, row2_v, stage_v,
              segblk_v, sems, q):
    i32 = jnp.int32
    f32 = jnp.float32
    b = q // _NR
    i = q - b * _NR
    lane_f = lax.iota(i32, 16).astype(f32)

    cps = [pltpu.make_async_copy(x_hbm.at[b, 0, pl.ds(i * _BS + r, 1)],
                                 rows_v[r], sems.at[0]) for r in range(_BS)]
    cp1 = pltpu.make_async_copy(x_hbm.at[b, 1, pl.ds(i * _BS + 8, 1)],
                                row1_v, sems.at[1])
    cp2 = pltpu.make_async_copy(x_hbm.at[b, 2, pl.ds(i * _BS + 8, 1)],
                                row2_v, sems.at[2])
    for c in cps:
        c.start()
    cp1.start()
    cp2.start()
    for c in cps:
        c.wait()
    cp1.wait()
    cp2.wait()

    sv = jnp.zeros((16,), f32)
    sv2 = jnp.zeros((16,), f32)
    rc = jnp.zeros((16,), f32)
    gc = jnp.zeros((16,), f32)
    bc = jnp.zeros((16,), f32)
    for j in range(_NR):
        acc = jnp.zeros((16,), f32)
        acc2 = jnp.zeros((16,), f32)
        for r in range(_BS):
            v = rows_v[r][0, pl.ds(j * _BS, _BS)]
            acc = acc + v
            acc2 = acc2 + v * v
        hot = jnp.maximum(1.0 - jnp.abs(lane_f - float(j)), 0.0)
        sv = sv + _extract_sum(acc) * hot
        sv2 = sv2 + _extract_sum(acc2) * hot
        rc = rc + rows_v[8][0, pl.ds(j * _BS, _BS)][8] * hot
        gc = gc + row1_v[0, pl.ds(j * _BS, _BS)][8] * hot
        bc = bc + row2_v[0, pl.ds(j * _BS, _BS)][8] * hot

    stage_v[pl.ds(0, 16)] = sv
    stage_v[pl.ds(16, 16)] = sv2
    stage_v[pl.ds(32, 16)] = rc
    stage_v[pl.ds(48, 16)] = gc
    stage_v[pl.ds(64, 16)] = bc
    pltpu.sync_copy(stage_v, sums_hbm.at[q])

    for j in range(_NR):
        vec = jnp.full((16,), i * _NR + j, i32)
        for r in range(_BS):
            segblk_v[pl.ds(r * _W + j * _BS, _BS)] = vec
    pltpu.sync_copy(segblk_v, seg_hbm.at[b, pl.ds(i * _BS * _W, _BS * _W)])


def _sc_feature_body(x_hbm, sums_hbm, seg_hbm, *refs):
    rows_v = list(refs[0:16])
    row1_v, row2_v, stage_v, segblk_v, sems = refs[16:21]
    wid = lax.axis_index("s") * 2 + lax.axis_index("c")
    _sc_strip(x_hbm, sums_hbm, seg_hbm, rows_v, row1_v, row2_v, stage_v,
              segblk_v, sems, wid)

    @pl.when(wid < _Q - _NWORKERS)
    def _():
        _sc_strip(x_hbm, sums_hbm, seg_hbm, rows_v, row1_v, row2_v, stage_v,
                  segblk_v, sems, wid + _NWORKERS)


def _make_sc_feature():
    return functools.partial(
        pl.kernel,
        out_type=(
            jax.ShapeDtypeStruct((_Q, 80), jnp.float32),
            jax.ShapeDtypeStruct((_B, _H * _W), jnp.int32),
        ),
        mesh=plsc.VectorSubcoreMesh(core_axis_name="c", subcore_axis_name="s",
                                    num_cores=2, num_subcores=16),
        scratch_types=[pltpu.VMEM((1, _W), jnp.float32) for _ in range(18)]
                      + [pltpu.VMEM((80,), jnp.float32),
                         pltpu.VMEM((_BS * _W,), jnp.int32),
                         pltpu.SemaphoreType.DMA((3,))],
    )(_sc_feature_body)


def _tc_mlp_kernel(sums_hbm, W1_hbm, W2_hbm, W3_hbm, W4_hbm, W5_hbm, vec_hbm,
                   y_hbm,
                   sums_v, w1_v, w2_v, w3_v, w4_v, w5_v, vec_v, y_v, sems):
    f32 = jnp.float32

    cp_sums = pltpu.make_async_copy(sums_hbm, sums_v, sems.at[0])
    cp_vec = pltpu.make_async_copy(vec_hbm, vec_v, sems.at[6])
    cp_w = [pltpu.make_async_copy(src, dst, sems.at[1 + i])
            for i, (src, dst) in enumerate(
                [(W1_hbm, w1_v), (W2_hbm, w2_v), (W3_hbm, w3_v),
                 (W4_hbm, w4_v), (W5_hbm, w5_v)])]
    cp_sums.start()
    cp_vec.start()
    for c in cp_w:
        c.start()

    def mm(a, b):
        return jnp.dot(a, b, preferred_element_type=f32)

    OneQ = (_iota((_Q, _NR, _Q), 0) == _iota((_Q, _NR, _Q), 2)
            ).astype(f32).reshape(_ROWS, _Q)
    OneJ = (_iota((_Q, _NR, 16), 1) == _iota((_Q, _NR, 16), 2)
            ).astype(f32).reshape(_ROWS, 16)

    def to_col(grid):
        return jnp.sum(mm(OneQ, grid) * OneJ, axis=1, keepdims=True)

    cp_sums.wait()
    S = sums_v[...]
    sv = to_col(S[:, 0:16])
    sv2 = to_col(S[:, 16:32])
    rcol = to_col(S[:, 32:48])
    gcol = to_col(S[:, 48:64])
    bcol = to_col(S[:, 64:80])

    mean = sv * (1.0 / _NPIX)
    var = jnp.maximum(sv2 - _NPIX * mean * mean, 0.0) * (1.0 / (_NPIX - 1))
    std = jnp.sqrt(var + 1e-12)

    xcen = (_iota((_Q, _NR, 1), 1) * _BS).astype(f32).reshape(_ROWS, 1) + 7.5
    ycen = ((_iota((_B, _NR, _NR, 1), 1) * _BS).astype(f32)
            .reshape(_ROWS, 1) + 7.5)

    cols = [xcen, ycen, mean, mean, mean, std, std, std, rcol, gcol, bcol]
    lane = _iota((_ROWS, 16), 1)
    X = jnp.zeros((_ROWS, 16), f32)
    for k, col in enumerate(cols):
        X = X + col * (lane == k).astype(f32)
    X = X[:, :_CH[0]]

    cp_vec.wait()
    vecs = vec_v[...]
    for li, wv in enumerate([w1_v, w2_v, w3_v, w4_v, w5_v]):
        cp_w[li].wait()
        Wt = wv[...]
        cout, off = _CH[li + 1], _VOFF[li]
        br = vecs[0:1, off:off + cout]
        gr = vecs[1:2, off:off + cout]
        ber = vecs[2:3, off:off + cout]
        y = lax.dot_general(X, Wt, (((1,), (1,)), ((), ())),
                            preferred_element_type=f32) + br
        mu = jnp.mean(y, axis=0, keepdims=True)
        d = y - mu
        var = jnp.mean(d * d, axis=0, keepdims=True)
        y = d * (gr / jnp.sqrt(var + 1e-5)) + ber
        if li < 4:
            y = jnp.maximum(y, 0.0)
        X = y
    y_v[...] = X
    cp_y = pltpu.make_async_copy(y_v, y_hbm, sems.at[7])
    cp_y.start()
    cp_y.wait()


def kernel(x, org_x, W1, b1, g1, be1, W2, b2, g2, be2, W3, b3, g3, be3,
           W4, b4, g4, be4, W5, b5, g5, be5):
    del org_x
    f32 = jnp.float32

    vecs = jnp.stack([jnp.concatenate([b1, b2, b3, b4, b5]),
                      jnp.concatenate([g1, g2, g3, g4, g5]),
                      jnp.concatenate([be1, be2, be3, be4, be5])])

    sums, segflat = _make_sc_feature()(x)
    seg = segflat.reshape(_B, _H, _W)

    any_spec = pl.BlockSpec(memory_space=pl.ANY)
    ymat = pl.pallas_call(
        _tc_mlp_kernel,
        in_specs=[any_spec] * 7,
        out_specs=any_spec,
        out_shape=jax.ShapeDtypeStruct((_ROWS, _CH[5]), f32),
        scratch_shapes=[
            pltpu.VMEM((_Q, 80), f32),
            pltpu.VMEM((_CH[1], _CH[0]), f32),
            pltpu.VMEM((_CH[2], _CH[1]), f32),
            pltpu.VMEM((_CH[3], _CH[2]), f32),
            pltpu.VMEM((_CH[4], _CH[3]), f32),
            pltpu.VMEM((_CH[5], _CH[4]), f32),
            pltpu.VMEM((3, _VTOT), f32),
            pltpu.VMEM((_ROWS, _CH[5]), f32),
            pltpu.SemaphoreType.DMA((8,)),
        ],
    )(sums, W1, W2, W3, W4, W5, vecs)

    y = ymat.reshape(_B, _S, -1).transpose(0, 2, 1).reshape(_B, -1, _NR, _NR)
    return (seg, y)


# speedup vs baseline: 1.0519x; 1.0519x over previous
import functools

import jax
import jax.numpy as jnp
from jax import lax
from jax.experimental import pallas as pl
from jax.experimental.pallas import tpu as pltpu
from jax.experimental.pallas import tpu_sc as plsc

_B, _H, _W, _NR = 4, 224, 224, 14
_BS = _H // _NR
_S = _NR * _NR
_ROWS = _B * _S
_Q = _B * _NR
_NPIX = _H * _W
_CH = [11, 96, 192, 384, 768, 768]
_VOFF = [0, 96, 288, 672, 1440]
_VTOT = 2208
_NWORKERS = 32


def _iota(shape, dim):
    return lax.broadcasted_iota(jnp.int32, shape, dim)


def _extract_sum(v):
    parts = [v[k] for k in range(16)]
    while len(parts) > 1:
        parts = [parts[i] + parts[i + 1] for i in range(0, len(parts), 2)]
    return parts[0]


def _sc_fetch(x_hbm, rows_v, row1_v, row2_v, sems, q):
    b = q // _NR
    i = q - b * _NR
    cps = [pltpu.make_async_copy(x_hbm.at[b, 0, pl.ds(i * _BS + r, 1)],
                                 rows_v[r], sems.at[0]) for r in range(_BS)]
    cps.append(pltpu.make_async_copy(x_hbm.at[b, 1, pl.ds(i * _BS + 8, 1)],
                                     row1_v, sems.at[0]))
    cps.append(pltpu.make_async_copy(x_hbm.at[b, 2, pl.ds(i * _BS + 8, 1)],
                                     row2_v, sems.at[0]))
    for c in cps:
        c.start()


def _sc_seg_write(seg_hbm, segblk_v, q):
    i32 = jnp.int32
    b = q // _NR
    i = q - b * _NR
    for j in range(_NR):
        vec = jnp.full((16,), i * _NR + j, i32)
        for r in range(_BS):
            segblk_v[pl.ds(r * _W + j * _BS, _BS)] = vec
    pltpu.sync_copy(segblk_v, seg_hbm.at[b, pl.ds(i * _BS * _W, _BS * _W)])


def _sc_sums(sums_hbm, rows_v, row1_v, row2_v, stage_v, q):
    f32 = jnp.float32
    lane_f = lax.iota(jnp.int32, 16).astype(f32)

    sv = jnp.zeros((16,), f32)
    sv2 = jnp.zeros((16,), f32)
    rc = jnp.zeros((16,), f32)
    gc = jnp.zeros((16,), f32)
    bc = jnp.zeros((16,), f32)
    for j in range(_NR):
        acc = jnp.zeros((16,), f32)
        acc2 = jnp.zeros((16,), f32)
        for r in range(_BS):
            v = rows_v[r][0, pl.ds(j * _BS, _BS)]
            acc = acc + v
            acc2 = acc2 + v * v
        hot = jnp.maximum(1.0 - jnp.abs(lane_f - float(j)), 0.0)
        sv = sv + _extract_sum(acc) * hot
        sv2 = sv2 + _extract_sum(acc2) * hot
        rc = rc + rows_v[8][0, pl.ds(j * _BS, _BS)][8] * hot
        gc = gc + row1_v[0, pl.ds(j * _BS, _BS)][8] * hot
        bc = bc + row2_v[0, pl.ds(j * _BS, _BS)][8] * hot

    stage_v[pl.ds(0, 16)] = sv
    stage_v[pl.ds(16, 16)] = sv2
    stage_v[pl.ds(32, 16)] = rc
    stage_v[pl.ds(48, 16)] = gc
    stage_v[pl.ds(64, 16)] = bc
    pltpu.sync_copy(stage_v, sums_hbm.at[q])


def _sc_drain(x_hbm, rows_v, row1_v, row2_v, sems, q):
    b = q // _NR
    i = q - b * _NR
    cps = [pltpu.make_async_copy(x_hbm.at[b, 0, pl.ds(i * _BS + r, 1)],
                                 rows_v[r], sems.at[0]) for r in range(_BS)]
    cps.append(pltpu.make_async_copy(x_hbm.at[b, 1, pl.ds(i * _BS + 8, 1)],
                                     row1_v, sems.at[0]))
    cps.append(pltpu.make_async_copy(x_hbm.at[b, 2, pl.ds(i * _BS + 8, 1)],
                                     row2_v, sems.at[0]))
    for c in cps:
        c.wait()


def _sc_feature_body(x_hbm, sums_hbm, seg_hbm, *refs):
    rows_a = list(refs[0:16])
    row1_a, row2_a = refs[16], refs[17]
    rows_b = list(refs[18:34])
    row1_b, row2_b = refs[34], refs[35]
    stage_v, segblk_v, sems = refs[36:39]
    wid = lax.axis_index("s") * 2 + lax.axis_index("c")

    _sc_fetch(x_hbm, rows_a, row1_a, row2_a, sems, wid)
    second = wid < _Q - _NWORKERS

    @pl.when(second)
    def _():
        _sc_fetch(x_hbm, rows_b, row1_b, row2_b, sems, wid + _NWORKERS)

    _sc_seg_write(seg_hbm, segblk_v, wid)

    @pl.when(second)
    def _():
        _sc_seg_write(seg_hbm, segblk_v, wid + _NWORKERS)

    _sc_drain(x_hbm, rows_a, row1_a, row2_a, sems, wid)
    _sc_sums(sums_hbm, rows_a, row1_a, row2_a, stage_v, wid)

    @pl.when(second)
    def _():
        _sc_drain(x_hbm, rows_b, row1_b, row2_b, sems, wid + _NWORKERS)
        _sc_sums(sums_hbm, rows_b, row1_b, row2_b, stage_v,
                 wid + _NWORKERS)


def _make_sc_feature():
    return functools.partial(
        pl.kernel,
        out_type=(
            jax.ShapeDtypeStruct((_Q, 80), jnp.float32),
            jax.ShapeDtypeStruct((_B, _H * _W), jnp.int32),
        ),
        mesh=plsc.VectorSubcoreMesh(core_axis_name="c", subcore_axis_name="s",
                                    num_cores=2, num_subcores=16),
        scratch_types=[pltpu.VMEM((1, _W), jnp.float32) for _ in range(36)]
                      + [pltpu.VMEM((80,), jnp.float32),
                         pltpu.VMEM((_BS * _W,), jnp.int32),
                         pltpu.SemaphoreType.DMA((1,))],
    )(_sc_feature_body)


def _tc_mlp_kernel(sums_hbm, W1_hbm, W2_hbm, W3_hbm, W4_hbm, W5_hbm, vec_hbm,
                   y_hbm,
                   sums_v, w1_v, w2_v, w3_v, w4_v, w5_v, vec_v, y_v, sems):
    f32 = jnp.float32

    cp_sums = pltpu.make_async_copy(sums_hbm, sums_v, sems.at[0])
    cp_vec = pltpu.make_async_copy(vec_hbm, vec_v, sems.at[6])
    cp_w = [pltpu.make_async_copy(src, dst, sems.at[1 + i])
            for i, (src, dst) in enumerate(
                [(W1_hbm, w1_v), (W2_hbm, w2_v), (W3_hbm, w3_v),
                 (W4_hbm, w4_v), (W5_hbm, w5_v)])]
    cp_sums.start()
    cp_vec.start()
    for c in cp_w:
        c.start()

    def mm(a, b):
        return jnp.dot(a, b, preferred_element_type=f32)

    OneQ = (_iota((_Q, _NR, _Q), 0) == _iota((_Q, _NR, _Q), 2)
            ).astype(f32).reshape(_ROWS, _Q)
    OneJ = (_iota((_Q, _NR, 16), 1) == _iota((_Q, _NR, 16), 2)
            ).astype(f32).reshape(_ROWS, 16)

    def to_col(grid):
        return jnp.sum(mm(OneQ, grid) * OneJ, axis=1, keepdims=True)

    cp_sums.wait()
    S = sums_v[...]
    sv = to_col(S[:, 0:16])
    sv2 = to_col(S[:, 16:32])
    rcol = to_col(S[:, 32:48])
    gcol = to_col(S[:, 48:64])
    bcol = to_col(S[:, 64:80])

    mean = sv * (1.0 / _NPIX)
    var = jnp.maximum(sv2 - _NPIX * mean * mean, 0.0) * (1.0 / (_NPIX - 1))
    std = jnp.sqrt(var + 1e-12)

    xcen = (_iota((_Q, _NR, 1), 1) * _BS).astype(f32).reshape(_ROWS, 1) + 7.5
    ycen = ((_iota((_B, _NR, _NR, 1), 1) * _BS).astype(f32)
            .reshape(_ROWS, 1) + 7.5)

    cols = [xcen, ycen, mean, mean, mean, std, std, std, rcol, gcol, bcol]
    lane = _iota((_ROWS, 16), 1)
    X = jnp.zeros((_ROWS, 16), f32)
    for k, col in enumerate(cols):
        X = X + col * (lane == k).astype(f32)
    X = X[:, :_CH[0]]

    cp_vec.wait()
    vecs = vec_v[...]
    for li, wv in enumerate([w1_v, w2_v, w3_v, w4_v, w5_v]):
        cp_w[li].wait()
        Wt = wv[...]
        cout, off = _CH[li + 1], _VOFF[li]
        br = vecs[0:1, off:off + cout]
        gr = vecs[1:2, off:off + cout]
        ber = vecs[2:3, off:off + cout]
        y = lax.dot_general(X, Wt, (((1,), (1,)), ((), ())),
                            preferred_element_type=f32) + br
        mu = jnp.mean(y, axis=0, keepdims=True)
        d = y - mu
        var = jnp.mean(d * d, axis=0, keepdims=True)
        y = d * (gr / jnp.sqrt(var + 1e-5)) + ber
        if li < 4:
            y = jnp.maximum(y, 0.0)
        X = y
    y_v[...] = X
    cp_y = pltpu.make_async_copy(y_v, y_hbm, sems.at[7])
    cp_y.start()
    cp_y.wait()


def kernel(x, org_x, W1, b1, g1, be1, W2, b2, g2, be2, W3, b3, g3, be3,
           W4, b4, g4, be4, W5, b5, g5, be5):
    del org_x
    f32 = jnp.float32

    vecs = jnp.stack([jnp.concatenate([b1, b2, b3, b4, b5]),
                      jnp.concatenate([g1, g2, g3, g4, g5]),
                      jnp.concatenate([be1, be2, be3, be4, be5])])

    sums, segflat = _make_sc_feature()(x)
    seg = segflat.reshape(_B, _H, _W)

    any_spec = pl.BlockSpec(memory_space=pl.ANY)
    ymat = pl.pallas_call(
        _tc_mlp_kernel,
        in_specs=[any_spec] * 7,
        out_specs=any_spec,
        out_shape=jax.ShapeDtypeStruct((_ROWS, _CH[5]), f32),
        scratch_shapes=[
            pltpu.VMEM((_Q, 80), f32),
            pltpu.VMEM((_CH[1], _CH[0]), f32),
            pltpu.VMEM((_CH[2], _CH[1]), f32),
            pltpu.VMEM((_CH[3], _CH[2]), f32),
            pltpu.VMEM((_CH[4], _CH[3]), f32),
            pltpu.VMEM((_CH[5], _CH[4]), f32),
            pltpu.VMEM((3, _VTOT), f32),
            pltpu.VMEM((_ROWS, _CH[5]), f32),
            pltpu.SemaphoreType.DMA((8,)),
        ],
    )(sums, W1, W2, W3, W4, W5, vecs)

    y = ymat.reshape(_B, _S, -1).transpose(0, 2, 1).reshape(_B, -1, _NR, _NR)
    return (seg, y)


# speedup vs baseline: 1.0559x; 1.0038x over previous
import functools

import jax
import jax.numpy as jnp
from jax import lax
from jax.experimental import pallas as pl
from jax.experimental.pallas import tpu as pltpu
from jax.experimental.pallas import tpu_sc as plsc

_B, _H, _W, _NR = 4, 224, 224, 14
_BS = _H // _NR
_S = _NR * _NR
_ROWS = _B * _S
_Q = _B * _NR
_NPIX = _H * _W
_CH = [11, 96, 192, 384, 768, 768]
_VOFF = [0, 96, 288, 672, 1440]
_VTOT = 2208
_NWORKERS = 32


def _iota(shape, dim):
    return lax.broadcasted_iota(jnp.int32, shape, dim)


def _extract_sum(v):
    parts = [v[k] for k in range(16)]
    while len(parts) > 1:
        parts = [parts[i] + parts[i + 1] for i in range(0, len(parts), 2)]
    return parts[0]


def _sc_fetch(x_hbm, rows_v, row1_v, row2_v, sems, q):
    b = q // _NR
    i = q - b * _NR
    cps = [pltpu.make_async_copy(x_hbm.at[b, 0, pl.ds(i * _BS + r, 1)],
                                 rows_v[r], sems.at[0]) for r in range(_BS)]
    cps.append(pltpu.make_async_copy(x_hbm.at[b, 1, pl.ds(i * _BS + 8, 1)],
                                     row1_v, sems.at[0]))
    cps.append(pltpu.make_async_copy(x_hbm.at[b, 2, pl.ds(i * _BS + 8, 1)],
                                     row2_v, sems.at[0]))
    for c in cps:
        c.start()


def _sc_seg_write(seg_hbm, segblk_v, q):
    i32 = jnp.int32
    b = q // _NR
    i = q - b * _NR
    for j in range(_NR):
        vec = jnp.full((16,), i * _NR + j, i32)
        for r in range(_BS):
            segblk_v[pl.ds(r * _W + j * _BS, _BS)] = vec
    pltpu.sync_copy(segblk_v, seg_hbm.at[b, pl.ds(i * _BS * _W, _BS * _W)])


def _sc_sums(sums_hbm, rows_v, row1_v, row2_v, stage_v, q):
    f32 = jnp.float32
    lane_f = lax.iota(jnp.int32, 16).astype(f32)

    sv = jnp.zeros((16,), f32)
    sv2 = jnp.zeros((16,), f32)
    rc = jnp.zeros((16,), f32)
    gc = jnp.zeros((16,), f32)
    bc = jnp.zeros((16,), f32)
    for j in range(_NR):
        acc = jnp.zeros((16,), f32)
        acc2 = jnp.zeros((16,), f32)
        for r in range(_BS):
            v = rows_v[r][0, pl.ds(j * _BS, _BS)]
            acc = acc + v
            acc2 = acc2 + v * v
        hot = jnp.maximum(1.0 - jnp.abs(lane_f - float(j)), 0.0)
        sv = sv + _extract_sum(acc) * hot
        sv2 = sv2 + _extract_sum(acc2) * hot
        rc = rc + rows_v[8][0, pl.ds(j * _BS, _BS)][8] * hot
        gc = gc + row1_v[0, pl.ds(j * _BS, _BS)][8] * hot
        bc = bc + row2_v[0, pl.ds(j * _BS, _BS)][8] * hot

    stage_v[pl.ds(0, 16)] = sv
    stage_v[pl.ds(16, 16)] = sv2
    stage_v[pl.ds(32, 16)] = rc
    stage_v[pl.ds(48, 16)] = gc
    stage_v[pl.ds(64, 16)] = bc
    pltpu.sync_copy(stage_v, sums_hbm.at[q])


def _sc_drain(x_hbm, rows_v, row1_v, row2_v, sems, q):
    b = q // _NR
    i = q - b * _NR
    cps = [pltpu.make_async_copy(x_hbm.at[b, 0, pl.ds(i * _BS + r, 1)],
                                 rows_v[r], sems.at[0]) for r in range(_BS)]
    cps.append(pltpu.make_async_copy(x_hbm.at[b, 1, pl.ds(i * _BS + 8, 1)],
                                     row1_v, sems.at[0]))
    cps.append(pltpu.make_async_copy(x_hbm.at[b, 2, pl.ds(i * _BS + 8, 1)],
                                     row2_v, sems.at[0]))
    for c in cps:
        c.wait()


def _sc_feature_body(x_hbm, sums_hbm, seg_hbm, *refs):
    rows_a = list(refs[0:16])
    row1_a, row2_a = refs[16], refs[17]
    rows_b = list(refs[18:34])
    row1_b, row2_b = refs[34], refs[35]
    stage_v, segblk_v, sems = refs[36:39]
    wid = lax.axis_index("s") * 2 + lax.axis_index("c")

    _sc_fetch(x_hbm, rows_a, row1_a, row2_a, sems, wid)
    second = wid < _Q - _NWORKERS

    @pl.when(second)
    def _():
        _sc_fetch(x_hbm, rows_b, row1_b, row2_b, sems, wid + _NWORKERS)

    _sc_seg_write(seg_hbm, segblk_v, wid)

    @pl.when(second)
    def _():
        _sc_seg_write(seg_hbm, segblk_v, wid + _NWORKERS)

    _sc_drain(x_hbm, rows_a, row1_a, row2_a, sems, wid)
    _sc_sums(sums_hbm, rows_a, row1_a, row2_a, stage_v, wid)

    @pl.when(second)
    def _():
        _sc_drain(x_hbm, rows_b, row1_b, row2_b, sems, wid + _NWORKERS)
        _sc_sums(sums_hbm, rows_b, row1_b, row2_b, stage_v,
                 wid + _NWORKERS)


def _make_sc_feature():
    return functools.partial(
        pl.kernel,
        out_type=(
            jax.ShapeDtypeStruct((_Q, 80), jnp.float32),
            jax.ShapeDtypeStruct((_B, _H * _W), jnp.int32),
        ),
        mesh=plsc.VectorSubcoreMesh(core_axis_name="c", subcore_axis_name="s",
                                    num_cores=2, num_subcores=16),
        scratch_types=[pltpu.VMEM((1, _W), jnp.float32) for _ in range(36)]
                      + [pltpu.VMEM((80,), jnp.float32),
                         pltpu.VMEM((_BS * _W,), jnp.int32),
                         pltpu.SemaphoreType.DMA((1,))],
    )(_sc_feature_body)


def _tc_mlp_kernel(sums_hbm, W1_hbm, W2_hbm, W3_hbm, W4_hbm, W5_hbm, vec_hbm,
                   y_hbm,
                   sums_v, w1_v, w2_v, w3_v, w4_v, w5_v, vec_v, y_v, sems):
    f32 = jnp.float32

    cp_sums = pltpu.make_async_copy(sums_hbm, sums_v, sems.at[0])
    cp_vec = pltpu.make_async_copy(vec_hbm, vec_v, sems.at[6])
    cp_w = [pltpu.make_async_copy(src, dst, sems.at[1 + i])
            for i, (src, dst) in enumerate(
                [(W1_hbm, w1_v), (W2_hbm, w2_v), (W3_hbm, w3_v),
                 (W4_hbm, w4_v), (W5_hbm, w5_v)])]
    cp_sums.start()
    cp_vec.start()
    for c in cp_w:
        c.start()

    def mm(a, b):
        return jnp.dot(a, b, preferred_element_type=f32)

    OneQ = (_iota((_Q, _NR, _Q), 0) == _iota((_Q, _NR, _Q), 2)
            ).astype(f32).reshape(_ROWS, _Q)
    OneJ = (_iota((_Q, _NR, 16), 1) == _iota((_Q, _NR, 16), 2)
            ).astype(f32).reshape(_ROWS, 16)

    def to_col(grid):
        return jnp.sum(mm(OneQ, grid) * OneJ, axis=1, keepdims=True)

    cp_sums.wait()
    S = sums_v[...]
    sv = to_col(S[:, 0:16])
    sv2 = to_col(S[:, 16:32])
    rcol = to_col(S[:, 32:48])
    gcol = to_col(S[:, 48:64])
    bcol = to_col(S[:, 64:80])

    mean = sv * (1.0 / _NPIX)
    var = jnp.maximum(sv2 - _NPIX * mean * mean, 0.0) * (1.0 / (_NPIX - 1))
    std = jnp.sqrt(var + 1e-12)

    xcen = (_iota((_Q, _NR, 1), 1) * _BS).astype(f32).reshape(_ROWS, 1) + 7.5
    ycen = ((_iota((_B, _NR, _NR, 1), 1) * _BS).astype(f32)
            .reshape(_ROWS, 1) + 7.5)

    cols = [xcen, ycen, mean, mean, mean, std, std, std, rcol, gcol, bcol]
    lane = _iota((_ROWS, 16), 1)
    X = jnp.zeros((_ROWS, 16), f32)
    for k, col in enumerate(cols):
        X = X + col * (lane == k).astype(f32)
    X = X[:, :_CH[0]]

    cp_vec.wait()
    vecs = vec_v[...]
    for li, wv in enumerate([w1_v, w2_v, w3_v, w4_v, w5_v]):
        cp_w[li].wait()
        Wt = wv[...]
        cout, off = _CH[li + 1], _VOFF[li]
        br = vecs[0:1, off:off + cout]
        gr = vecs[1:2, off:off + cout]
        ber = vecs[2:3, off:off + cout]
        Xin = X.astype(jnp.bfloat16) if li >= 3 else X
        y = lax.dot_general(Xin, Wt, (((1,), (1,)), ((), ())),
                            preferred_element_type=f32) + br
        mu = jnp.mean(y, axis=0, keepdims=True)
        d = y - mu
        var = jnp.mean(d * d, axis=0, keepdims=True)
        y = d * (gr / jnp.sqrt(var + 1e-5)) + ber
        if li < 4:
            y = jnp.maximum(y, 0.0)
        X = y
    y_v[...] = X
    cp_y = pltpu.make_async_copy(y_v, y_hbm, sems.at[7])
    cp_y.start()
    cp_y.wait()


def kernel(x, org_x, W1, b1, g1, be1, W2, b2, g2, be2, W3, b3, g3, be3,
           W4, b4, g4, be4, W5, b5, g5, be5):
    del org_x
    f32 = jnp.float32

    vecs = jnp.stack([jnp.concatenate([b1, b2, b3, b4, b5]),
                      jnp.concatenate([g1, g2, g3, g4, g5]),
                      jnp.concatenate([be1, be2, be3, be4, be5])])

    sums, segflat = _make_sc_feature()(x)
    seg = segflat.reshape(_B, _H, _W)

    any_spec = pl.BlockSpec(memory_space=pl.ANY)
    ymat = pl.pallas_call(
        _tc_mlp_kernel,
        in_specs=[any_spec] * 7,
        out_specs=any_spec,
        out_shape=jax.ShapeDtypeStruct((_ROWS, _CH[5]), f32),
        scratch_shapes=[
            pltpu.VMEM((_Q, 80), f32),
            pltpu.VMEM((_CH[1], _CH[0]), f32),
            pltpu.VMEM((_CH[2], _CH[1]), f32),
            pltpu.VMEM((_CH[3], _CH[2]), f32),
            pltpu.VMEM((_CH[4], _CH[3]), jnp.bfloat16),
            pltpu.VMEM((_CH[5], _CH[4]), jnp.bfloat16),
            pltpu.VMEM((3, _VTOT), f32),
            pltpu.VMEM((_ROWS, _CH[5]), f32),
            pltpu.SemaphoreType.DMA((8,)),
        ],
    )(sums, W1, W2, W3, W4.astype(jnp.bfloat16), W5.astype(jnp.bfloat16), vecs)

    y = ymat.reshape(_B, _S, -1).transpose(0, 2, 1).reshape(_B, -1, _NR, _NR)
    return (seg, y)
